# baseline (speedup 1.0000x reference)
import dataclasses
import functools

import numpy as np
import jax
import jax.numpy as jnp
from jax import lax
from jax.experimental import pallas as pl
from jax.experimental.pallas import tpu as pltpu
from jax.experimental.pallas import tpu_sc as plsc

N_ROWS = 16384
N_COLS = 1000
SC_COLS = 256
_NUM_CORES = 2
_NUM_SUBCORES = 16
_NW = _NUM_CORES * _NUM_SUBCORES
_CHUNK = 64

TC_B = 2048
_TC_COL_STEPS = (1024 - SC_COLS) // 128

_AR16 = np.arange(16, dtype=np.int32)


def _sc_partial(x, row_base, rows):
    rpw = rows // _NW
    mesh = plsc.VectorSubcoreMesh(core_axis_name="c", subcore_axis_name="s")
    out_sd = [
        jax.ShapeDtypeStruct((rows,), jnp.float32),
        jax.ShapeDtypeStruct((rows,), jnp.int32),
    ]

    cp = pltpu.CompilerParams()
    if "needs_layout_passes" in pltpu.CompilerParams.__dataclass_fields__:
        cp = dataclasses.replace(cp, needs_layout_passes=False)

    @functools.partial(
        pl.kernel,
        mesh=mesh,
        out_type=out_sd,
        compiler_params=cp,
        scratch_types=[
            pltpu.VMEM((_CHUNK, 128), jnp.float32),
            pltpu.VMEM((_CHUNK, 128), jnp.float32),
            pltpu.VMEM((16, 16), jnp.float32),
            pltpu.VMEM((16, 16), jnp.int32),
            pltpu.VMEM((rpw,), jnp.float32),
            pltpu.VMEM((rpw,), jnp.int32),
        ],
    )
    def k(x_hbm, mv_hbm, mc_hbm, p0, p1, mvT, mcT, omv, omc):
        wid = lax.axis_index("s") * _NUM_CORES + lax.axis_index("c")
        ar16 = lax.broadcasted_iota(jnp.int32, (16,), 0)

        @pl.loop(0, rpw // _CHUNK)
        def _(chunk):
            local_lo = wid * rpw + chunk * _CHUNK
            lo = pl.multiple_of(row_base + local_lo, _CHUNK)
            pltpu.sync_copy(x_hbm.at[pl.ds(lo, _CHUNK), pl.ds(0, 128)], p0)
            pltpu.sync_copy(x_hbm.at[pl.ds(lo, _CHUNK), pl.ds(128, 128)], p1)

            @pl.loop(0, _CHUNK // 16)
            def _(g):
                @pl.loop(0, 16)
                def _(rr):
                    r = g * 16 + rr
                    mv = p0.at[r, pl.ds(0, 16)][...]
                    mc = jnp.zeros((16,), jnp.int32)
                    for k_ in range(1, 16):
                        src = p0 if k_ < 8 else p1
                        off = (k_ % 8) * 16
                        v = src.at[r, pl.ds(off, 16)][...]
                        upd = v > mv
                        mv = jnp.where(upd, v, mv)
                        mc = jnp.where(upd, jnp.full((16,), k_, jnp.int32), mc)
                    rvec = jnp.full((16,), rr, jnp.int32)
                    plsc.store_scatter(mvT, [ar16, rvec], mv)
                    plsc.store_scatter(mcT, [ar16, rvec], mc)

                bm = mvT.at[0, pl.ds(0, 16)][...]
                bc = mcT.at[0, pl.ds(0, 16)][...] * 16
                for l in range(1, 16):
                    v = mvT.at[l, pl.ds(0, 16)][...]
                    c = mcT.at[l, pl.ds(0, 16)][...] * 16 + l
                    upd = (v > bm) | ((v == bm) & (c < bc))
                    bm = jnp.where(upd, v, bm)
                    bc = jnp.where(upd, c, bc)
                obase = chunk * _CHUNK + g * 16
                omv.at[pl.ds(obase, 16)][...] = bm
                omc.at[pl.ds(obase, 16)][...] = bc

        dst = pl.ds(pl.multiple_of(wid * rpw, rpw), rpw)
        pltpu.sync_copy(omv, mv_hbm.at[dst])
        pltpu.sync_copy(omc, mc_hbm.at[dst])

    return k(x)


def _tc_body(x_ref, scmv_ref, scmc_ref, t_ref, o_ref, mv_s, mc_s):
    j = pl.program_id(1)
    b = x_ref.shape[0]

    @pl.when(j == 0)
    def _():
        mv_s[...] = jnp.full((b, 128), -jnp.inf, jnp.float32)
        mc_s[...] = jnp.zeros((b, 128), jnp.int32)

    xb = x_ref[...]
    colg = SC_COLS + j * 128 + lax.broadcasted_iota(jnp.int32, (b, 128), 1)
    xa = jnp.where(colg < N_COLS, xb, -jnp.inf)
    upd = xa > mv_s[...]
    mv_s[...] = jnp.where(upd, xa, mv_s[...])
    mc_s[...] = jnp.where(upd, colg, mc_s[...])

    @pl.when(j == _TC_COL_STEPS - 1)
    def _():
        scv = jnp.reshape(scmv_ref[...], (b, 1))
        scc = jnp.reshape(scmc_ref[...], (b, 1))
        m1 = jnp.max(mv_s[...], axis=1, keepdims=True)
        mm = jnp.maximum(m1, scv)
        c1 = jnp.min(
            jnp.where(mv_s[...] == mm, mc_s[...], N_COLS), axis=1, keepdims=True
        )
        c2 = jnp.where(scv == mm, scc, N_COLS)
        idx = jnp.minimum(c1, c2)
        cols = lax.broadcasted_iota(jnp.int32, (b, N_COLS), 1)
        onehot = (cols == idx).astype(jnp.float32)
        o_ref[...] = jnp.dot(onehot, t_ref[...], preferred_element_type=jnp.float32)


def _tc_combine(x, scmv, scmc, table):
    n, c = x.shape
    d = table.shape[1]
    grid = (n // TC_B, _TC_COL_STEPS)
    return pl.pallas_call(
        _tc_body,
        grid=grid,
        in_specs=[
            pl.BlockSpec((TC_B, 128), lambda i, j: (i, j + SC_COLS // 128)),
            pl.BlockSpec((TC_B,), lambda i, j: (i,)),
            pl.BlockSpec((TC_B,), lambda i, j: (i,)),
            pl.BlockSpec((c, d), lambda i, j: (0, 0)),
        ],
        out_specs=pl.BlockSpec((TC_B, d), lambda i, j: (i, 0)),
        out_shape=jax.ShapeDtypeStruct((n, d), jnp.float32),
        scratch_shapes=[
            pltpu.VMEM((TC_B, 128), jnp.float32),
            pltpu.VMEM((TC_B, 128), jnp.int32),
        ],
    )(x, scmv, scmc, table)


def kernel(x, table):
    scmv, scmc = _sc_partial(x, 0, N_ROWS)
    return _tc_combine(x, scmv, scmc, table)


# speedup vs baseline: 1.1157x; 1.1157x over previous
import dataclasses
import functools

import numpy as np
import jax
import jax.numpy as jnp
from jax import lax
from jax.experimental import pallas as pl
from jax.experimental.pallas import tpu as pltpu
from jax.experimental.pallas import tpu_sc as plsc

N_ROWS = 16384
N_COLS = 1000
SC_COLS = 512
_NUM_CORES = 2
_NUM_SUBCORES = 16
_NW = _NUM_CORES * _NUM_SUBCORES
_CHUNK = 64

TC_B = 2048


def _sc_partial(x, row_base, rows):
    rpw = rows // _NW
    nreg = SC_COLS // 16
    mesh = plsc.VectorSubcoreMesh(core_axis_name="c", subcore_axis_name="s")
    out_sd = [
        jax.ShapeDtypeStruct((rows,), jnp.float32),
        jax.ShapeDtypeStruct((rows,), jnp.int32),
    ]
    cp = pltpu.CompilerParams()
    if "needs_layout_passes" in pltpu.CompilerParams.__dataclass_fields__:
        cp = dataclasses.replace(cp, needs_layout_passes=False)

    @functools.partial(
        pl.kernel,
        mesh=mesh,
        out_type=out_sd,
        compiler_params=cp,
        scratch_types=[
            pltpu.VMEM((_CHUNK, 128), jnp.float32),
            pltpu.VMEM((_CHUNK, 128), jnp.float32),
            pltpu.VMEM((_CHUNK, 128), jnp.float32),
            pltpu.VMEM((_CHUNK, 128), jnp.float32),
            pltpu.VMEM((16, 16), jnp.float32),
            pltpu.VMEM((16, 16), jnp.int32),
            pltpu.VMEM((rpw,), jnp.float32),
            pltpu.VMEM((rpw,), jnp.int32),
        ],
    )
    def k(x_hbm, mv_hbm, mc_hbm, p0, p1, p2, p3, mvT, mcT, omv, omc):
        wid = lax.axis_index("s") * _NUM_CORES + lax.axis_index("c")
        ar16 = lax.broadcasted_iota(jnp.int32, (16,), 0)
        panels = [p0, p1, p2, p3]

        @pl.loop(0, rpw // _CHUNK)
        def _(chunk):
            local_lo = wid * rpw + chunk * _CHUNK
            lo = pl.multiple_of(row_base + local_lo, _CHUNK)
            for p_ in range(4):
                pltpu.sync_copy(
                    x_hbm.at[pl.ds(lo, _CHUNK), pl.ds(p_ * 128, 128)], panels[p_]
                )

            @pl.loop(0, _CHUNK // 16)
            def _(g):
                @pl.loop(0, 16)
                def _(rr):
                    r = g * 16 + rr
                    mv = p0.at[r, pl.ds(0, 16)][...]
                    mc = jnp.zeros((16,), jnp.int32)
                    for k_ in range(1, nreg):
                        src = panels[k_ // 8]
                        off = (k_ % 8) * 16
                        v = src.at[r, pl.ds(off, 16)][...]
                        upd = v > mv
                        mv = jnp.where(upd, v, mv)
                        mc = jnp.where(upd, jnp.full((16,), k_, jnp.int32), mc)
                    rvec = jnp.full((16,), rr, jnp.int32)
                    plsc.store_scatter(mvT, [ar16, rvec], mv)
                    plsc.store_scatter(mcT, [ar16, rvec], mc)

                bm = mvT.at[0, pl.ds(0, 16)][...]
                bc = mcT.at[0, pl.ds(0, 16)][...] * 16
                for l in range(1, 16):
                    v = mvT.at[l, pl.ds(0, 16)][...]
                    c = mcT.at[l, pl.ds(0, 16)][...] * 16 + l
                    upd = (v > bm) | ((v == bm) & (c < bc))
                    bm = jnp.where(upd, v, bm)
                    bc = jnp.where(upd, c, bc)
                obase = chunk * _CHUNK + g * 16
                omv.at[pl.ds(obase, 16)][...] = bm
                omc.at[pl.ds(obase, 16)][...] = bc

        dst = pl.ds(pl.multiple_of(wid * rpw, rpw), rpw)
        pltpu.sync_copy(omv, mv_hbm.at[dst])
        pltpu.sync_copy(omc, mc_hbm.at[dst])

    return k(x)


def _tc_body(x_ref, scmv_ref, scmc_ref, t_ref, o_ref):
    b = x_ref.shape[0]
    xb = x_ref[...]
    colg = SC_COLS + lax.broadcasted_iota(jnp.int32, (b, 512), 1)
    xa = jnp.where(colg < N_COLS, xb, -jnp.inf)
    m1 = jnp.max(xa, axis=1, keepdims=True)
    scv = jnp.reshape(scmv_ref[...], (b, 1))
    scc = jnp.reshape(scmc_ref[...], (b, 1))
    mm = jnp.maximum(m1, scv)
    c1 = jnp.min(jnp.where(xa == mm, colg, N_COLS), axis=1, keepdims=True)
    c2 = jnp.where(scv == mm, scc, N_COLS)
    idx = jnp.minimum(c1, c2)
    cols = lax.broadcasted_iota(jnp.int32, (b, N_COLS), 1)
    onehot = (cols == idx).astype(jnp.float32)
    o_ref[...] = jnp.dot(onehot, t_ref[...], preferred_element_type=jnp.float32)


def _tc_combine(x, scmv, scmc, table):
    n, c = x.shape
    d = table.shape[1]
    return pl.pallas_call(
        _tc_body,
        grid=(n // TC_B,),
        in_specs=[
            pl.BlockSpec((TC_B, 512), lambda i: (i, 1)),
            pl.BlockSpec((TC_B,), lambda i: (i,)),
            pl.BlockSpec((TC_B,), lambda i: (i,)),
            pl.BlockSpec((c, d), lambda i: (0, 0)),
        ],
        out_specs=pl.BlockSpec((TC_B, d), lambda i: (i, 0)),
        out_shape=jax.ShapeDtypeStruct((n, d), jnp.float32),
    )(x, scmv, scmc, table)


def kernel(x, table):
    scmv, scmc = _sc_partial(x, 0, N_ROWS)
    return _tc_combine(x, scmv, scmc, table)
